# baseline (speedup 1.0000x reference)
import functools
import math

import jax
import jax.numpy as jnp
from jax import lax
from jax.experimental import pallas as pl
from jax.experimental.pallas import tpu as pltpu
from jax.experimental.pallas import tpu_sc as plsc

_CHUNK = 128
_NW = 32


def _proj_body(a_ref, wq_ref, wk_ref, wv_ref, bq_ref, bk_ref, bv_ref,
               q_ref, kv_ref):
    a = a_ref[...]
    q = jnp.dot(a, wq_ref[...], preferred_element_type=jnp.float32) + bq_ref[...]
    k = jnp.dot(a, wk_ref[...], preferred_element_type=jnp.float32) + bk_ref[...]
    v = jnp.dot(a, wv_ref[...], preferred_element_type=jnp.float32) + bv_ref[...]
    q_ref[...] = q
    ku = lax.bitcast_convert_type(
        k.astype(jnp.bfloat16).astype(jnp.float32), jnp.uint32) >> 16
    vu = lax.bitcast_convert_type(
        v.astype(jnp.bfloat16).astype(jnp.float32), jnp.uint32) & jnp.uint32(0xFFFF0000)
    kv_ref[...] = lax.bitcast_convert_type(ku | vu, jnp.int32)


def _node_proj(atom, wqt, wkt, wvt, bq, bk, bv):
    n, d = atom.shape
    bn = 1000
    grid = (n // bn,)
    return pl.pallas_call(
        _proj_body,
        grid=grid,
        in_specs=[
            pl.BlockSpec((bn, d), lambda i: (i, 0)),
            pl.BlockSpec((d, d), lambda i: (0, 0)),
            pl.BlockSpec((d, d), lambda i: (0, 0)),
            pl.BlockSpec((d, d), lambda i: (0, 0)),
            pl.BlockSpec((1, d), lambda i: (0, 0)),
            pl.BlockSpec((1, d), lambda i: (0, 0)),
            pl.BlockSpec((1, d), lambda i: (0, 0)),
        ],
        out_specs=[
            pl.BlockSpec((bn, d), lambda i: (i, 0)),
            pl.BlockSpec((bn, d), lambda i: (i, 0)),
        ],
        out_shape=[
            jax.ShapeDtypeStruct((n, d), jnp.float32),
            jax.ShapeDtypeStruct((n, d), jnp.int32),
        ],
    )(atom, wqt, wkt, wvt, bq, bk, bv)


def _edge_math_body(has_prev, *refs):
    if has_prev:
        (qd_ref, kvs_ref, ef_ref, wet_ref, be_ref, _prev_ref,
         eo_ref, msg_ref) = refs
    else:
        qd_ref, kvs_ref, ef_ref, wet_ref, be_ref, eo_ref, msg_ref = refs
    qd = qd_ref[...]
    w = lax.bitcast_convert_type(kvs_ref[...], jnp.uint32)
    ks = lax.bitcast_convert_type(w << 16, jnp.float32)
    vs = lax.bitcast_convert_type(w & jnp.uint32(0xFFFF0000), jnp.float32)
    ef = ef_ref[...]
    d = qd_ref.shape[1]
    eproj = jnp.dot(ef, wet_ref[...], preferred_element_type=jnp.float32) + be_ref[...]
    t = qd * ks
    t = t - jnp.max(t, axis=-1, keepdims=True)
    p = jnp.exp(t)
    attn = p / jnp.sum(p, axis=-1, keepdims=True)
    attn = attn / jnp.sqrt(jnp.float32(d)) + eproj
    eo_ref[...] = ef + attn
    msg_ref[...] = attn * vs


def _edge_math(qd, kvs, ef, wet, be, si, prev_eo):
    e_s, d = qd.shape
    e_full = ef.shape[0]
    be_rows = 1600
    nblk = e_s // be_rows
    off = si
    in_specs = [
        pl.BlockSpec((be_rows, d), lambda i: (i, 0)),
        pl.BlockSpec((be_rows, d), lambda i: (i, 0)),
        pl.BlockSpec((be_rows, d), lambda i: (i + off, 0)),
        pl.BlockSpec((d, d), lambda i: (0, 0)),
        pl.BlockSpec((1, d), lambda i: (0, 0)),
    ]
    args = [qd, kvs, ef, wet, be]
    aliases = {}
    if prev_eo is not None:
        in_specs.append(pl.BlockSpec((8, d), lambda i: (0, 0)))
        args.append(prev_eo)
        aliases = {5: 0}
    return pl.pallas_call(
        functools.partial(_edge_math_body, prev_eo is not None),
        grid=(nblk,),
        in_specs=in_specs,
        out_specs=[
            pl.BlockSpec((be_rows, d), lambda i: (i + off, 0)),
            pl.BlockSpec((be_rows, d), lambda i: (i, 0)),
        ],
        out_shape=[
            jax.ShapeDtypeStruct((e_full, d), jnp.float32),
            jax.ShapeDtypeStruct((e_s, d), jnp.float32),
        ],
        input_output_aliases=aliases,
    )(*args)


def _combine_body(*refs):
    a_ref = refs[0]
    o_ref = refs[-1]
    acc = a_ref[...]
    for r in refs[1:-1]:
        acc = acc + r[0]
    o_ref[...] = acc


def _combine(atom, partials_list):
    n, d = atom.shape
    bn = 1000
    in_specs = [pl.BlockSpec((bn, d), lambda i: (i, 0))]
    args = [atom]
    for prt in partials_list:
        in_specs.append(pl.BlockSpec((1, bn, d), lambda i: (0, i, 0)))
        in_specs.append(pl.BlockSpec((1, bn, d), lambda i: (1, i, 0)))
        args.extend([prt, prt])
    return pl.pallas_call(
        _combine_body,
        grid=(n // bn,),
        in_specs=in_specs,
        out_specs=pl.BlockSpec((bn, d), lambda i: (i, 0)),
        out_shape=jax.ShapeDtypeStruct((n, d), jnp.float32),
    )(*args)


_SC_MESH = plsc.VectorSubcoreMesh(core_axis_name="c", subcore_axis_name="s")


def _gather(q, kv, ids):
    n, dq = q.shape
    dkv = kv.shape[1]
    nchunks = ids.shape[0]
    e = nchunks * _CHUNK
    tmax = -(-nchunks // _NW)
    ntrips = (tmax + 2 + 3) // 3

    @functools.partial(
        pl.kernel,
        mesh=_SC_MESH,
        out_type=[
            jax.ShapeDtypeStruct((e, dq), q.dtype),
            jax.ShapeDtypeStruct((e, dkv), jnp.int32),
        ],
        scratch_types=[
            pltpu.VMEM((3, 2, _CHUNK), jnp.int32),
            pltpu.VMEM((3, _CHUNK, dq), q.dtype),
            pltpu.VMEM((3, _CHUNK, dkv), jnp.int32),
            pltpu.SemaphoreType.DMA,
            pltpu.SemaphoreType.DMA,
            pltpu.SemaphoreType.DMA,
            pltpu.SemaphoreType.DMA,
            pltpu.SemaphoreType.DMA,
            pltpu.SemaphoreType.DMA,
            pltpu.SemaphoreType.DMA,
            pltpu.SemaphoreType.DMA,
            pltpu.SemaphoreType.DMA,
        ],
    )
    def k(q_hbm, kv_hbm, ids_hbm, qd_out, kvs_out,
          idxb, qbuf, kvbuf, si0, si1, si2, sg0, sg1, sg2, sw0, sw1, sw2):
        w = lax.axis_index("s") * 2 + lax.axis_index("c")
        si = (si0, si1, si2)
        sg = (sg0, sg1, sg2)
        sw = (sw0, sw1, sw2)

        def chunk_of(j):
            return w + _NW * j

        def ok(j):
            return (j >= 0) & (chunk_of(j) < nchunks)

        def issue_idx(j, b):
            @pl.when(ok(j))
            def _():
                pltpu.async_copy(ids_hbm.at[chunk_of(j)], idxb.at[b], si[b])

        issue_idx(0, 0)

        def phase(j, b):
            pb = (b + 2) % 3

            @pl.when(ok(j - 3))
            def _():
                pltpu.make_async_copy(
                    qbuf.at[b], qd_out.at[pl.ds(0, _CHUNK)], sw[b]).wait()
                pltpu.make_async_copy(
                    kvbuf.at[b], kvs_out.at[pl.ds(0, _CHUNK)], sw[b]).wait()

            @pl.when(ok(j))
            def _():
                pltpu.make_async_copy(ids_hbm.at[0], idxb.at[b], si[b]).wait()
                pltpu.async_copy(q_hbm.at[idxb.at[b, 0]], qbuf.at[b], sg[b])
                pltpu.async_copy(kv_hbm.at[idxb.at[b, 1]], kvbuf.at[b], sg[b])

            @pl.when(ok(j - 1))
            def _():
                pltpu.make_async_copy(
                    q_hbm.at[pl.ds(0, _CHUNK)], qbuf.at[pb], sg[pb]).wait()
                pltpu.make_async_copy(
                    kv_hbm.at[pl.ds(0, _CHUNK)], kvbuf.at[pb], sg[pb]).wait()
                c = chunk_of(j - 1)
                pltpu.async_copy(
                    qbuf.at[pb], qd_out.at[pl.ds(c * _CHUNK, _CHUNK)], sw[pb])
                pltpu.async_copy(
                    kvbuf.at[pb], kvs_out.at[pl.ds(c * _CHUNK, _CHUNK)], sw[pb])

            issue_idx(j + 1, (b + 1) % 3)

        def body(jt, carry):
            phase(3 * jt, 0)
            phase(3 * jt + 1, 1)
            phase(3 * jt + 2, 2)
            return carry

        lax.fori_loop(0, ntrips, body, 0)

    return k(q, kv, ids)


def _scatter(message, dst3d, zeros_pd):
    e, d = message.shape
    npad = zeros_pd.shape[0]
    nchunks = dst3d.shape[0]
    tmax = -(-nchunks // _NW)
    rows_per_tile = npad // 16
    ntrips = (tmax + 2 + 3) // 3

    @functools.partial(
        pl.kernel,
        mesh=_SC_MESH,
        out_type=jax.ShapeDtypeStruct((2, npad, d), jnp.float32),
        scratch_types=[
            pltpu.VMEM_SHARED((npad, d), jnp.float32),
            pltpu.VMEM((3, _CHUNK), jnp.int32),
            pltpu.VMEM((3, _CHUNK, d), jnp.float32),
            pltpu.SemaphoreType.DMA,
            pltpu.SemaphoreType.DMA,
            pltpu.SemaphoreType.DMA,
            pltpu.SemaphoreType.DMA,
            pltpu.SemaphoreType.DMA,
            pltpu.SemaphoreType.DMA,
            pltpu.SemaphoreType.DMA,
            pltpu.SemaphoreType.DMA,
            pltpu.SemaphoreType.DMA,
        ],
    )
    def k(msg_hbm, dst_hbm, zero_hbm, part_out, acc_sh, dstb, msgb,
          si0, si1, si2, sl0, sl1, sl2, ss0, ss1, ss2):
        c = lax.axis_index("c")
        s = lax.axis_index("s")
        w = s * 2 + c
        r0 = s * rows_per_tile
        si = (si0, si1, si2)
        sl = (sl0, sl1, sl2)
        ss = (ss0, ss1, ss2)

        pltpu.sync_copy(zero_hbm.at[pl.ds(r0, rows_per_tile)],
                        acc_sh.at[pl.ds(r0, rows_per_tile)])
        plsc.subcore_barrier()

        def chunk_of(j):
            return w + _NW * j

        def ok(j):
            return (j >= 0) & (chunk_of(j) < nchunks)

        def phase(j, b):
            pb = (b + 2) % 3

            @pl.when(ok(j - 3))
            def _():
                pltpu.make_async_copy(
                    msgb.at[b], acc_sh.at[pl.ds(0, _CHUNK)], ss[b]).wait()

            @pl.when(ok(j))
            def _():
                chunk = chunk_of(j)
                pltpu.async_copy(dst_hbm.at[chunk, 0], dstb.at[b], si[b])
                pltpu.async_copy(
                    msg_hbm.at[pl.ds(chunk * _CHUNK, _CHUNK)], msgb.at[b], sl[b])

            @pl.when(ok(j - 1))
            def _():
                pltpu.make_async_copy(
                    dst_hbm.at[0, 0], dstb.at[pb], si[pb]).wait()
                pltpu.make_async_copy(
                    msg_hbm.at[pl.ds(0, _CHUNK)], msgb.at[pb], sl[pb]).wait()
                pltpu.async_copy(
                    msgb.at[pb], acc_sh.at[dstb.at[pb]], ss[pb], add=True)

        def body(jt, carry):
            phase(3 * jt, 0)
            phase(3 * jt + 1, 1)
            phase(3 * jt + 2, 2)
            return carry

        lax.fori_loop(0, ntrips, body, 0)
        plsc.subcore_barrier()
        pltpu.sync_copy(acc_sh.at[pl.ds(r0, rows_per_tile)],
                        part_out.at[c, pl.ds(r0, rows_per_tile)])

    return k(message, dst3d, zeros_pd)


def kernel(atom_feature, edge_feature, edge_index, Wq, bq, Wk, bk, Wv, bv, We, be):
    n, d = atom_feature.shape
    e = edge_feature.shape[0]
    nchunks = e // _CHUNK
    npad = ((n + 127) // 128) * 128
    slice_chunks = (850, 850, 800)
    assert sum(slice_chunks) == nchunks

    q, kvp = _node_proj(atom_feature, Wq.T, Wk.T, Wv.T,
                        bq.reshape(1, d), bk.reshape(1, d), bv.reshape(1, d))
    dst3d = edge_index[1].reshape(nchunks, 1, _CHUNK)
    ids = jnp.stack([edge_index[1].reshape(nchunks, _CHUNK),
                     edge_index[0].reshape(nchunks, _CHUNK)], axis=1)
    zeros_pd = jnp.zeros((npad, d), jnp.float32)

    prev_eo = None
    partials_list = []
    edge_out = None
    c0 = 0
    for cs in slice_chunks:
        ids_s = ids[c0:c0 + cs]
        qd_s, kvs_s = _gather(q, kvp, ids_s)
        edge_out, msg_s = _edge_math(qd_s, kvs_s, edge_feature, We.T,
                                     be.reshape(1, d),
                                     c0 * _CHUNK // 1600, prev_eo)
        prev_eo = edge_out
        partials_list.append(_scatter(msg_s, dst3d[c0:c0 + cs], zeros_pd))
        c0 += cs
    atom_out = _combine(atom_feature, partials_list)
    return (atom_out, edge_out)


# speedup vs baseline: 1.0348x; 1.0348x over previous
import functools
import math

import jax
import jax.numpy as jnp
from jax import lax
from jax.experimental import pallas as pl
from jax.experimental.pallas import tpu as pltpu
from jax.experimental.pallas import tpu_sc as plsc

_CHUNK = 128
_NW = 32


def _proj_body(a_ref, wq_ref, wk_ref, wv_ref, bq_ref, bk_ref, bv_ref,
               q_ref, kv_ref):
    a = a_ref[...]
    q = jnp.dot(a, wq_ref[...], preferred_element_type=jnp.float32) + bq_ref[...]
    k = jnp.dot(a, wk_ref[...], preferred_element_type=jnp.float32) + bk_ref[...]
    v = jnp.dot(a, wv_ref[...], preferred_element_type=jnp.float32) + bv_ref[...]
    q_ref[...] = q
    ku = lax.bitcast_convert_type(
        k.astype(jnp.bfloat16).astype(jnp.float32), jnp.uint32) >> 16
    vu = lax.bitcast_convert_type(
        v.astype(jnp.bfloat16).astype(jnp.float32), jnp.uint32) & jnp.uint32(0xFFFF0000)
    kv_ref[...] = lax.bitcast_convert_type(ku | vu, jnp.int32)


def _node_proj(atom, wqt, wkt, wvt, bq, bk, bv):
    n, d = atom.shape
    bn = 1000
    grid = (n // bn,)
    return pl.pallas_call(
        _proj_body,
        grid=grid,
        in_specs=[
            pl.BlockSpec((bn, d), lambda i: (i, 0)),
            pl.BlockSpec((d, d), lambda i: (0, 0)),
            pl.BlockSpec((d, d), lambda i: (0, 0)),
            pl.BlockSpec((d, d), lambda i: (0, 0)),
            pl.BlockSpec((1, d), lambda i: (0, 0)),
            pl.BlockSpec((1, d), lambda i: (0, 0)),
            pl.BlockSpec((1, d), lambda i: (0, 0)),
        ],
        out_specs=[
            pl.BlockSpec((bn, d), lambda i: (i, 0)),
            pl.BlockSpec((bn, d), lambda i: (i, 0)),
        ],
        out_shape=[
            jax.ShapeDtypeStruct((n, d), jnp.float32),
            jax.ShapeDtypeStruct((n, d), jnp.int32),
        ],
    )(atom, wqt, wkt, wvt, bq, bk, bv)


def _edge_math_body(has_prev, *refs):
    if has_prev:
        (qd_ref, kvs_ref, ef_ref, wet_ref, be_ref, _prev_ref,
         eo_ref, msg_ref) = refs
    else:
        qd_ref, kvs_ref, ef_ref, wet_ref, be_ref, eo_ref, msg_ref = refs
    qd = qd_ref[...]
    w = lax.bitcast_convert_type(kvs_ref[...], jnp.uint32)
    ks = lax.bitcast_convert_type(w << 16, jnp.float32)
    vs = lax.bitcast_convert_type(w & jnp.uint32(0xFFFF0000), jnp.float32)
    ef = ef_ref[...]
    d = qd_ref.shape[1]
    eproj = jnp.dot(ef, wet_ref[...], preferred_element_type=jnp.float32) + be_ref[...]
    t = qd * ks
    t = t - jnp.max(t, axis=-1, keepdims=True)
    p = jnp.exp(t)
    attn = p / jnp.sum(p, axis=-1, keepdims=True)
    attn = attn / jnp.sqrt(jnp.float32(d)) + eproj
    eo_ref[...] = ef + attn
    msg_ref[...] = attn * vs


def _edge_math(qd, kvs, ef, wet, be, si, prev_eo):
    e_s, d = qd.shape
    e_full = ef.shape[0]
    be_rows = 8000
    nblk = e_s // be_rows
    off = si
    in_specs = [
        pl.BlockSpec((be_rows, d), lambda i: (i, 0)),
        pl.BlockSpec((be_rows, d), lambda i: (i, 0)),
        pl.BlockSpec((be_rows, d), lambda i: (i + off, 0)),
        pl.BlockSpec((d, d), lambda i: (0, 0)),
        pl.BlockSpec((1, d), lambda i: (0, 0)),
    ]
    args = [qd, kvs, ef, wet, be]
    aliases = {}
    if prev_eo is not None:
        in_specs.append(pl.BlockSpec((8, d), lambda i: (0, 0)))
        args.append(prev_eo)
        aliases = {5: 0}
    return pl.pallas_call(
        functools.partial(_edge_math_body, prev_eo is not None),
        grid=(nblk,),
        in_specs=in_specs,
        out_specs=[
            pl.BlockSpec((be_rows, d), lambda i: (i + off, 0)),
            pl.BlockSpec((be_rows, d), lambda i: (i, 0)),
        ],
        out_shape=[
            jax.ShapeDtypeStruct((e_full, d), jnp.float32),
            jax.ShapeDtypeStruct((e_s, d), jnp.float32),
        ],
        input_output_aliases=aliases,
    )(*args)


def _combine_body(*refs):
    a_ref = refs[0]
    o_ref = refs[-1]
    acc = a_ref[...]
    for r in refs[1:-1]:
        acc = acc + r[0]
    o_ref[...] = acc


def _combine(atom, partials_list):
    n, d = atom.shape
    bn = 1000
    in_specs = [pl.BlockSpec((bn, d), lambda i: (i, 0))]
    args = [atom]
    for prt in partials_list:
        in_specs.append(pl.BlockSpec((1, bn, d), lambda i: (0, i, 0)))
        in_specs.append(pl.BlockSpec((1, bn, d), lambda i: (1, i, 0)))
        args.extend([prt, prt])
    return pl.pallas_call(
        _combine_body,
        grid=(n // bn,),
        in_specs=in_specs,
        out_specs=pl.BlockSpec((bn, d), lambda i: (i, 0)),
        out_shape=jax.ShapeDtypeStruct((n, d), jnp.float32),
    )(*args)


_SC_MESH = plsc.VectorSubcoreMesh(core_axis_name="c", subcore_axis_name="s")


def _gather(q, kv, ids):
    n, dq = q.shape
    dkv = kv.shape[1]
    nchunks = ids.shape[0]
    e = nchunks * _CHUNK
    tmax = -(-nchunks // _NW)
    ntrips = (tmax + 2 + 3) // 3

    @functools.partial(
        pl.kernel,
        mesh=_SC_MESH,
        out_type=[
            jax.ShapeDtypeStruct((e, dq), q.dtype),
            jax.ShapeDtypeStruct((e, dkv), jnp.int32),
        ],
        scratch_types=[
            pltpu.VMEM((3, 2, _CHUNK), jnp.int32),
            pltpu.VMEM((3, _CHUNK, dq), q.dtype),
            pltpu.VMEM((3, _CHUNK, dkv), jnp.int32),
            pltpu.SemaphoreType.DMA,
            pltpu.SemaphoreType.DMA,
            pltpu.SemaphoreType.DMA,
            pltpu.SemaphoreType.DMA,
            pltpu.SemaphoreType.DMA,
            pltpu.SemaphoreType.DMA,
            pltpu.SemaphoreType.DMA,
            pltpu.SemaphoreType.DMA,
            pltpu.SemaphoreType.DMA,
        ],
    )
    def k(q_hbm, kv_hbm, ids_hbm, qd_out, kvs_out,
          idxb, qbuf, kvbuf, si0, si1, si2, sg0, sg1, sg2, sw0, sw1, sw2):
        w = lax.axis_index("s") * 2 + lax.axis_index("c")
        si = (si0, si1, si2)
        sg = (sg0, sg1, sg2)
        sw = (sw0, sw1, sw2)

        def chunk_of(j):
            return w + _NW * j

        def ok(j):
            return (j >= 0) & (chunk_of(j) < nchunks)

        def issue_idx(j, b):
            @pl.when(ok(j))
            def _():
                pltpu.async_copy(ids_hbm.at[chunk_of(j)], idxb.at[b], si[b])

        issue_idx(0, 0)

        def phase(j, b):
            pb = (b + 2) % 3

            @pl.when(ok(j - 3))
            def _():
                pltpu.make_async_copy(
                    qbuf.at[b], qd_out.at[pl.ds(0, _CHUNK)], sw[b]).wait()
                pltpu.make_async_copy(
                    kvbuf.at[b], kvs_out.at[pl.ds(0, _CHUNK)], sw[b]).wait()

            @pl.when(ok(j))
            def _():
                pltpu.make_async_copy(ids_hbm.at[0], idxb.at[b], si[b]).wait()
                pltpu.async_copy(q_hbm.at[idxb.at[b, 0]], qbuf.at[b], sg[b])
                pltpu.async_copy(kv_hbm.at[idxb.at[b, 1]], kvbuf.at[b], sg[b])

            @pl.when(ok(j - 1))
            def _():
                pltpu.make_async_copy(
                    q_hbm.at[pl.ds(0, _CHUNK)], qbuf.at[pb], sg[pb]).wait()
                pltpu.make_async_copy(
                    kv_hbm.at[pl.ds(0, _CHUNK)], kvbuf.at[pb], sg[pb]).wait()
                c = chunk_of(j - 1)
                pltpu.async_copy(
                    qbuf.at[pb], qd_out.at[pl.ds(c * _CHUNK, _CHUNK)], sw[pb])
                pltpu.async_copy(
                    kvbuf.at[pb], kvs_out.at[pl.ds(c * _CHUNK, _CHUNK)], sw[pb])

            issue_idx(j + 1, (b + 1) % 3)

        def body(jt, carry):
            phase(3 * jt, 0)
            phase(3 * jt + 1, 1)
            phase(3 * jt + 2, 2)
            return carry

        lax.fori_loop(0, ntrips, body, 0)

    return k(q, kv, ids)


def _scatter(message, dst3d, zeros_pd):
    e, d = message.shape
    npad = zeros_pd.shape[0]
    nchunks = dst3d.shape[0]
    tmax = -(-nchunks // _NW)
    rows_per_tile = npad // 16
    ntrips = (tmax + 2 + 3) // 3

    @functools.partial(
        pl.kernel,
        mesh=_SC_MESH,
        out_type=jax.ShapeDtypeStruct((2, npad, d), jnp.float32),
        scratch_types=[
            pltpu.VMEM_SHARED((npad, d), jnp.float32),
            pltpu.VMEM((3, _CHUNK), jnp.int32),
            pltpu.VMEM((3, _CHUNK, d), jnp.float32),
            pltpu.SemaphoreType.DMA,
            pltpu.SemaphoreType.DMA,
            pltpu.SemaphoreType.DMA,
            pltpu.SemaphoreType.DMA,
            pltpu.SemaphoreType.DMA,
            pltpu.SemaphoreType.DMA,
            pltpu.SemaphoreType.DMA,
            pltpu.SemaphoreType.DMA,
            pltpu.SemaphoreType.DMA,
        ],
    )
    def k(msg_hbm, dst_hbm, zero_hbm, part_out, acc_sh, dstb, msgb,
          si0, si1, si2, sl0, sl1, sl2, ss0, ss1, ss2):
        c = lax.axis_index("c")
        s = lax.axis_index("s")
        w = s * 2 + c
        r0 = s * rows_per_tile
        si = (si0, si1, si2)
        sl = (sl0, sl1, sl2)
        ss = (ss0, ss1, ss2)

        pltpu.sync_copy(zero_hbm.at[pl.ds(r0, rows_per_tile)],
                        acc_sh.at[pl.ds(r0, rows_per_tile)])
        plsc.subcore_barrier()

        def chunk_of(j):
            return w + _NW * j

        def ok(j):
            return (j >= 0) & (chunk_of(j) < nchunks)

        def phase(j, b):
            pb = (b + 2) % 3

            @pl.when(ok(j - 3))
            def _():
                pltpu.make_async_copy(
                    msgb.at[b], acc_sh.at[pl.ds(0, _CHUNK)], ss[b]).wait()

            @pl.when(ok(j))
            def _():
                chunk = chunk_of(j)
                pltpu.async_copy(dst_hbm.at[chunk, 0], dstb.at[b], si[b])
                pltpu.async_copy(
                    msg_hbm.at[pl.ds(chunk * _CHUNK, _CHUNK)], msgb.at[b], sl[b])

            @pl.when(ok(j - 1))
            def _():
                pltpu.make_async_copy(
                    dst_hbm.at[0, 0], dstb.at[pb], si[pb]).wait()
                pltpu.make_async_copy(
                    msg_hbm.at[pl.ds(0, _CHUNK)], msgb.at[pb], sl[pb]).wait()
                pltpu.async_copy(
                    msgb.at[pb], acc_sh.at[dstb.at[pb]], ss[pb], add=True)

        def body(jt, carry):
            phase(3 * jt, 0)
            phase(3 * jt + 1, 1)
            phase(3 * jt + 2, 2)
            return carry

        lax.fori_loop(0, ntrips, body, 0)
        plsc.subcore_barrier()
        pltpu.sync_copy(acc_sh.at[pl.ds(r0, rows_per_tile)],
                        part_out.at[c, pl.ds(r0, rows_per_tile)])

    return k(message, dst3d, zeros_pd)


def kernel(atom_feature, edge_feature, edge_index, Wq, bq, Wk, bk, Wv, bv, We, be):
    n, d = atom_feature.shape
    e = edge_feature.shape[0]
    nchunks = e // _CHUNK
    npad = ((n + 127) // 128) * 128
    slice_chunks = (1250, 1250)
    assert sum(slice_chunks) == nchunks

    q, kvp = _node_proj(atom_feature, Wq.T, Wk.T, Wv.T,
                        bq.reshape(1, d), bk.reshape(1, d), bv.reshape(1, d))
    dst3d = edge_index[1].reshape(nchunks, 1, _CHUNK)
    ids = jnp.stack([edge_index[1].reshape(nchunks, _CHUNK),
                     edge_index[0].reshape(nchunks, _CHUNK)], axis=1)
    zeros_pd = jnp.zeros((npad, d), jnp.float32)

    prev_eo = None
    partials_list = []
    edge_out = None
    c0 = 0
    for cs in slice_chunks:
        ids_s = ids[c0:c0 + cs]
        qd_s, kvs_s = _gather(q, kvp, ids_s)
        edge_out, msg_s = _edge_math(qd_s, kvs_s, edge_feature, We.T,
                                     be.reshape(1, d),
                                     c0 * _CHUNK // 8000, prev_eo)
        prev_eo = edge_out
        partials_list.append(_scatter(msg_s, dst3d[c0:c0 + cs], zeros_pd))
        c0 += cs
    atom_out = _combine(atom_feature, partials_list)
    return (atom_out, edge_out)


# speedup vs baseline: 1.0349x; 1.0001x over previous
import functools
import math

import jax
import jax.numpy as jnp
from jax import lax
from jax.experimental import pallas as pl
from jax.experimental.pallas import tpu as pltpu
from jax.experimental.pallas import tpu_sc as plsc

_CHUNK = 128
_NW = 32


def _proj_body(a_ref, wq_ref, wk_ref, wv_ref, bq_ref, bk_ref, bv_ref,
               q_ref, kv_ref):
    a = a_ref[...]
    q = jnp.dot(a, wq_ref[...], preferred_element_type=jnp.float32) + bq_ref[...]
    k = jnp.dot(a, wk_ref[...], preferred_element_type=jnp.float32) + bk_ref[...]
    v = jnp.dot(a, wv_ref[...], preferred_element_type=jnp.float32) + bv_ref[...]
    q_ref[...] = q
    ku = lax.bitcast_convert_type(
        k.astype(jnp.bfloat16).astype(jnp.float32), jnp.uint32) >> 16
    vu = lax.bitcast_convert_type(
        v.astype(jnp.bfloat16).astype(jnp.float32), jnp.uint32) & jnp.uint32(0xFFFF0000)
    kv_ref[...] = lax.bitcast_convert_type(ku | vu, jnp.int32)


def _node_proj(atom, wqt, wkt, wvt, bq, bk, bv):
    n, d = atom.shape
    bn = 1000
    grid = (n // bn,)
    return pl.pallas_call(
        _proj_body,
        grid=grid,
        in_specs=[
            pl.BlockSpec((bn, d), lambda i: (i, 0)),
            pl.BlockSpec((d, d), lambda i: (0, 0)),
            pl.BlockSpec((d, d), lambda i: (0, 0)),
            pl.BlockSpec((d, d), lambda i: (0, 0)),
            pl.BlockSpec((1, d), lambda i: (0, 0)),
            pl.BlockSpec((1, d), lambda i: (0, 0)),
            pl.BlockSpec((1, d), lambda i: (0, 0)),
        ],
        out_specs=[
            pl.BlockSpec((bn, d), lambda i: (i, 0)),
            pl.BlockSpec((bn, d), lambda i: (i, 0)),
        ],
        out_shape=[
            jax.ShapeDtypeStruct((n, d), jnp.float32),
            jax.ShapeDtypeStruct((n, d), jnp.int32),
        ],
    )(atom, wqt, wkt, wvt, bq, bk, bv)


def _edge_math_body(has_prev, *refs):
    if has_prev:
        (qd_ref, kvs_ref, ef_ref, wet_ref, be_ref, _prev_ref,
         eo_ref, msg_ref) = refs
    else:
        qd_ref, kvs_ref, ef_ref, wet_ref, be_ref, eo_ref, msg_ref = refs
    qd = qd_ref[...]
    w = lax.bitcast_convert_type(kvs_ref[...], jnp.uint32)
    ks = lax.bitcast_convert_type(w << 16, jnp.float32)
    vs = lax.bitcast_convert_type(w & jnp.uint32(0xFFFF0000), jnp.float32)
    ef = ef_ref[...]
    d = qd_ref.shape[1]
    eproj = jnp.dot(ef, wet_ref[...], preferred_element_type=jnp.float32) + be_ref[...]
    p = jnp.exp(qd * ks)
    inv = (1.0 / math.sqrt(d)) / jnp.sum(p, axis=-1, keepdims=True)
    attn = p * inv + eproj
    eo_ref[...] = ef + attn
    msg_ref[...] = attn * vs


def _edge_math(qd, kvs, ef, wet, be, si, prev_eo):
    e_s, d = qd.shape
    e_full = ef.shape[0]
    be_rows = 8000
    nblk = e_s // be_rows
    off = si
    in_specs = [
        pl.BlockSpec((be_rows, d), lambda i: (i, 0)),
        pl.BlockSpec((be_rows, d), lambda i: (i, 0)),
        pl.BlockSpec((be_rows, d), lambda i: (i + off, 0)),
        pl.BlockSpec((d, d), lambda i: (0, 0)),
        pl.BlockSpec((1, d), lambda i: (0, 0)),
    ]
    args = [qd, kvs, ef, wet, be]
    aliases = {}
    if prev_eo is not None:
        in_specs.append(pl.BlockSpec((8, d), lambda i: (0, 0)))
        args.append(prev_eo)
        aliases = {5: 0}
    return pl.pallas_call(
        functools.partial(_edge_math_body, prev_eo is not None),
        grid=(nblk,),
        in_specs=in_specs,
        out_specs=[
            pl.BlockSpec((be_rows, d), lambda i: (i + off, 0)),
            pl.BlockSpec((be_rows, d), lambda i: (i, 0)),
        ],
        out_shape=[
            jax.ShapeDtypeStruct((e_full, d), jnp.float32),
            jax.ShapeDtypeStruct((e_s, d), jnp.float32),
        ],
        input_output_aliases=aliases,
    )(*args)


def _combine_body(*refs):
    a_ref = refs[0]
    o_ref = refs[-1]
    acc = a_ref[...]
    for r in refs[1:-1]:
        acc = acc + r[0]
    o_ref[...] = acc


def _combine(atom, partials_list):
    n, d = atom.shape
    bn = 1000
    in_specs = [pl.BlockSpec((bn, d), lambda i: (i, 0))]
    args = [atom]
    for prt in partials_list:
        in_specs.append(pl.BlockSpec((1, bn, d), lambda i: (0, i, 0)))
        in_specs.append(pl.BlockSpec((1, bn, d), lambda i: (1, i, 0)))
        args.extend([prt, prt])
    return pl.pallas_call(
        _combine_body,
        grid=(n // bn,),
        in_specs=in_specs,
        out_specs=pl.BlockSpec((bn, d), lambda i: (i, 0)),
        out_shape=jax.ShapeDtypeStruct((n, d), jnp.float32),
    )(*args)


_SC_MESH = plsc.VectorSubcoreMesh(core_axis_name="c", subcore_axis_name="s")


def _gather(q, kv, ids):
    n, dq = q.shape
    dkv = kv.shape[1]
    nchunks = ids.shape[0]
    e = nchunks * _CHUNK
    tmax = -(-nchunks // _NW)
    ntrips = (tmax + 2 + 3) // 3

    @functools.partial(
        pl.kernel,
        mesh=_SC_MESH,
        out_type=[
            jax.ShapeDtypeStruct((e, dq), q.dtype),
            jax.ShapeDtypeStruct((e, dkv), jnp.int32),
        ],
        scratch_types=[
            pltpu.VMEM((3, 2, _CHUNK), jnp.int32),
            pltpu.VMEM((3, _CHUNK, dq), q.dtype),
            pltpu.VMEM((3, _CHUNK, dkv), jnp.int32),
            pltpu.SemaphoreType.DMA,
            pltpu.SemaphoreType.DMA,
            pltpu.SemaphoreType.DMA,
            pltpu.SemaphoreType.DMA,
            pltpu.SemaphoreType.DMA,
            pltpu.SemaphoreType.DMA,
            pltpu.SemaphoreType.DMA,
            pltpu.SemaphoreType.DMA,
            pltpu.SemaphoreType.DMA,
        ],
    )
    def k(q_hbm, kv_hbm, ids_hbm, qd_out, kvs_out,
          idxb, qbuf, kvbuf, si0, si1, si2, sg0, sg1, sg2, sw0, sw1, sw2):
        w = lax.axis_index("s") * 2 + lax.axis_index("c")
        si = (si0, si1, si2)
        sg = (sg0, sg1, sg2)
        sw = (sw0, sw1, sw2)

        def chunk_of(j):
            return w + _NW * j

        def ok(j):
            return (j >= 0) & (chunk_of(j) < nchunks)

        def issue_idx(j, b):
            @pl.when(ok(j))
            def _():
                pltpu.async_copy(ids_hbm.at[chunk_of(j)], idxb.at[b], si[b])

        issue_idx(0, 0)

        def phase(j, b):
            pb = (b + 2) % 3

            @pl.when(ok(j - 3))
            def _():
                pltpu.make_async_copy(
                    qbuf.at[b], qd_out.at[pl.ds(0, _CHUNK)], sw[b]).wait()
                pltpu.make_async_copy(
                    kvbuf.at[b], kvs_out.at[pl.ds(0, _CHUNK)], sw[b]).wait()

            @pl.when(ok(j))
            def _():
                pltpu.make_async_copy(ids_hbm.at[0], idxb.at[b], si[b]).wait()
                pltpu.async_copy(q_hbm.at[idxb.at[b, 0]], qbuf.at[b], sg[b])
                pltpu.async_copy(kv_hbm.at[idxb.at[b, 1]], kvbuf.at[b], sg[b])

            @pl.when(ok(j - 1))
            def _():
                pltpu.make_async_copy(
                    q_hbm.at[pl.ds(0, _CHUNK)], qbuf.at[pb], sg[pb]).wait()
                pltpu.make_async_copy(
                    kv_hbm.at[pl.ds(0, _CHUNK)], kvbuf.at[pb], sg[pb]).wait()
                c = chunk_of(j - 1)
                pltpu.async_copy(
                    qbuf.at[pb], qd_out.at[pl.ds(c * _CHUNK, _CHUNK)], sw[pb])
                pltpu.async_copy(
                    kvbuf.at[pb], kvs_out.at[pl.ds(c * _CHUNK, _CHUNK)], sw[pb])

            issue_idx(j + 1, (b + 1) % 3)

        def body(jt, carry):
            phase(3 * jt, 0)
            phase(3 * jt + 1, 1)
            phase(3 * jt + 2, 2)
            return carry

        lax.fori_loop(0, ntrips, body, 0)

    return k(q, kv, ids)


def _scatter(message, dst3d, zeros_pd):
    e, d = message.shape
    npad = zeros_pd.shape[0]
    nchunks = dst3d.shape[0]
    tmax = -(-nchunks // _NW)
    rows_per_tile = npad // 16
    ntrips = (tmax + 2 + 3) // 3

    @functools.partial(
        pl.kernel,
        mesh=_SC_MESH,
        out_type=jax.ShapeDtypeStruct((2, npad, d), jnp.float32),
        scratch_types=[
            pltpu.VMEM_SHARED((npad, d), jnp.float32),
            pltpu.VMEM((3, _CHUNK), jnp.int32),
            pltpu.VMEM((3, _CHUNK, d), jnp.float32),
            pltpu.SemaphoreType.DMA,
            pltpu.SemaphoreType.DMA,
            pltpu.SemaphoreType.DMA,
            pltpu.SemaphoreType.DMA,
            pltpu.SemaphoreType.DMA,
            pltpu.SemaphoreType.DMA,
            pltpu.SemaphoreType.DMA,
            pltpu.SemaphoreType.DMA,
            pltpu.SemaphoreType.DMA,
        ],
    )
    def k(msg_hbm, dst_hbm, zero_hbm, part_out, acc_sh, dstb, msgb,
          si0, si1, si2, sl0, sl1, sl2, ss0, ss1, ss2):
        c = lax.axis_index("c")
        s = lax.axis_index("s")
        w = s * 2 + c
        r0 = s * rows_per_tile
        si = (si0, si1, si2)
        sl = (sl0, sl1, sl2)
        ss = (ss0, ss1, ss2)

        pltpu.sync_copy(zero_hbm.at[pl.ds(r0, rows_per_tile)],
                        acc_sh.at[pl.ds(r0, rows_per_tile)])
        plsc.subcore_barrier()

        def chunk_of(j):
            return w + _NW * j

        def ok(j):
            return (j >= 0) & (chunk_of(j) < nchunks)

        def phase(j, b):
            pb = (b + 2) % 3

            @pl.when(ok(j - 3))
            def _():
                pltpu.make_async_copy(
                    msgb.at[b], acc_sh.at[pl.ds(0, _CHUNK)], ss[b]).wait()

            @pl.when(ok(j))
            def _():
                chunk = chunk_of(j)
                pltpu.async_copy(dst_hbm.at[chunk, 0], dstb.at[b], si[b])
                pltpu.async_copy(
                    msg_hbm.at[pl.ds(chunk * _CHUNK, _CHUNK)], msgb.at[b], sl[b])

            @pl.when(ok(j - 1))
            def _():
                pltpu.make_async_copy(
                    dst_hbm.at[0, 0], dstb.at[pb], si[pb]).wait()
                pltpu.make_async_copy(
                    msg_hbm.at[pl.ds(0, _CHUNK)], msgb.at[pb], sl[pb]).wait()
                pltpu.async_copy(
                    msgb.at[pb], acc_sh.at[dstb.at[pb]], ss[pb], add=True)

        def body(jt, carry):
            phase(3 * jt, 0)
            phase(3 * jt + 1, 1)
            phase(3 * jt + 2, 2)
            return carry

        lax.fori_loop(0, ntrips, body, 0)
        plsc.subcore_barrier()
        pltpu.sync_copy(acc_sh.at[pl.ds(r0, rows_per_tile)],
                        part_out.at[c, pl.ds(r0, rows_per_tile)])

    return k(message, dst3d, zeros_pd)


def kernel(atom_feature, edge_feature, edge_index, Wq, bq, Wk, bk, Wv, bv, We, be):
    n, d = atom_feature.shape
    e = edge_feature.shape[0]
    nchunks = e // _CHUNK
    npad = ((n + 127) // 128) * 128
    slice_chunks = (1250, 1250)
    assert sum(slice_chunks) == nchunks

    q, kvp = _node_proj(atom_feature, Wq.T, Wk.T, Wv.T,
                        bq.reshape(1, d), bk.reshape(1, d), bv.reshape(1, d))
    dst3d = edge_index[1].reshape(nchunks, 1, _CHUNK)
    ids = jnp.stack([edge_index[1].reshape(nchunks, _CHUNK),
                     edge_index[0].reshape(nchunks, _CHUNK)], axis=1)
    zeros_pd = jnp.zeros((npad, d), jnp.float32)

    prev_eo = None
    partials_list = []
    edge_out = None
    c0 = 0
    for cs in slice_chunks:
        ids_s = ids[c0:c0 + cs]
        qd_s, kvs_s = _gather(q, kvp, ids_s)
        edge_out, msg_s = _edge_math(qd_s, kvs_s, edge_feature, We.T,
                                     be.reshape(1, d),
                                     c0 * _CHUNK // 8000, prev_eo)
        prev_eo = edge_out
        partials_list.append(_scatter(msg_s, dst3d[c0:c0 + cs], zeros_pd))
        c0 += cs
    atom_out = _combine(atom_feature, partials_list)
    return (atom_out, edge_out)


# speedup vs baseline: 1.0417x; 1.0066x over previous
import functools
import math

import jax
import jax.numpy as jnp
from jax import lax
from jax.experimental import pallas as pl
from jax.experimental.pallas import tpu as pltpu
from jax.experimental.pallas import tpu_sc as plsc

_CHUNK = 128
_NW = 32


def _proj_body(a_ref, wq_ref, wk_ref, wv_ref, bq_ref, bk_ref, bv_ref,
               q_ref, kv_ref):
    a = a_ref[...]
    q = jnp.dot(a, wq_ref[...], preferred_element_type=jnp.float32) + bq_ref[...]
    k = jnp.dot(a, wk_ref[...], preferred_element_type=jnp.float32) + bk_ref[...]
    v = jnp.dot(a, wv_ref[...], preferred_element_type=jnp.float32) + bv_ref[...]
    q_ref[...] = q
    ku = lax.bitcast_convert_type(
        k.astype(jnp.bfloat16).astype(jnp.float32), jnp.uint32) >> 16
    vu = lax.bitcast_convert_type(
        v.astype(jnp.bfloat16).astype(jnp.float32), jnp.uint32) & jnp.uint32(0xFFFF0000)
    kv_ref[...] = lax.bitcast_convert_type(ku | vu, jnp.int32)


def _node_proj(atom, wqt, wkt, wvt, bq, bk, bv):
    n, d = atom.shape
    bn = 2000
    grid = (n // bn,)
    return pl.pallas_call(
        _proj_body,
        grid=grid,
        in_specs=[
            pl.BlockSpec((bn, d), lambda i: (i, 0)),
            pl.BlockSpec((d, d), lambda i: (0, 0)),
            pl.BlockSpec((d, d), lambda i: (0, 0)),
            pl.BlockSpec((d, d), lambda i: (0, 0)),
            pl.BlockSpec((1, d), lambda i: (0, 0)),
            pl.BlockSpec((1, d), lambda i: (0, 0)),
            pl.BlockSpec((1, d), lambda i: (0, 0)),
        ],
        out_specs=[
            pl.BlockSpec((bn, d), lambda i: (i, 0)),
            pl.BlockSpec((bn, d), lambda i: (i, 0)),
        ],
        out_shape=[
            jax.ShapeDtypeStruct((n, d), jnp.float32),
            jax.ShapeDtypeStruct((n, d), jnp.int32),
        ],
    )(atom, wqt, wkt, wvt, bq, bk, bv)


def _edge_math_body(has_prev, *refs):
    if has_prev:
        (qd_ref, kvs_ref, ef_ref, wet_ref, be_ref, _prev_ref,
         eo_ref, msg_ref) = refs
    else:
        qd_ref, kvs_ref, ef_ref, wet_ref, be_ref, eo_ref, msg_ref = refs
    qd = qd_ref[...]
    w = lax.bitcast_convert_type(kvs_ref[...], jnp.uint32)
    ks = lax.bitcast_convert_type(w << 16, jnp.float32)
    vs = lax.bitcast_convert_type(w & jnp.uint32(0xFFFF0000), jnp.float32)
    ef = ef_ref[...]
    d = qd_ref.shape[1]
    eproj = jnp.dot(ef, wet_ref[...], preferred_element_type=jnp.float32) + be_ref[...]
    p = jnp.exp(qd * ks)
    inv = (1.0 / math.sqrt(d)) / jnp.sum(p, axis=-1, keepdims=True)
    attn = p * inv + eproj
    eo_ref[...] = ef + attn
    msg_ref[...] = attn * vs


def _edge_math(qd, kvs, ef, wet, be, si, prev_eo):
    e_s, d = qd.shape
    e_full = ef.shape[0]
    be_rows = 8000
    nblk = e_s // be_rows
    off = si
    in_specs = [
        pl.BlockSpec((be_rows, d), lambda i: (i, 0)),
        pl.BlockSpec((be_rows, d), lambda i: (i, 0)),
        pl.BlockSpec((be_rows, d), lambda i: (i + off, 0)),
        pl.BlockSpec((d, d), lambda i: (0, 0)),
        pl.BlockSpec((1, d), lambda i: (0, 0)),
    ]
    args = [qd, kvs, ef, wet, be]
    aliases = {}
    if prev_eo is not None:
        in_specs.append(pl.BlockSpec((8, d), lambda i: (0, 0)))
        args.append(prev_eo)
        aliases = {5: 0}
    return pl.pallas_call(
        functools.partial(_edge_math_body, prev_eo is not None),
        grid=(nblk,),
        in_specs=in_specs,
        out_specs=[
            pl.BlockSpec((be_rows, d), lambda i: (i + off, 0)),
            pl.BlockSpec((be_rows, d), lambda i: (i, 0)),
        ],
        out_shape=[
            jax.ShapeDtypeStruct((e_full, d), jnp.float32),
            jax.ShapeDtypeStruct((e_s, d), jnp.float32),
        ],
        input_output_aliases=aliases,
    )(*args)


def _combine_body(*refs):
    a_ref = refs[0]
    o_ref = refs[-1]
    acc = a_ref[...]
    for r in refs[1:-1]:
        acc = acc + r[0]
    o_ref[...] = acc


def _combine(atom, partials_list):
    n, d = atom.shape
    bn = 1000
    in_specs = [pl.BlockSpec((bn, d), lambda i: (i, 0))]
    args = [atom]
    for prt in partials_list:
        in_specs.append(pl.BlockSpec((1, bn, d), lambda i: (0, i, 0)))
        in_specs.append(pl.BlockSpec((1, bn, d), lambda i: (1, i, 0)))
        args.extend([prt, prt])
    return pl.pallas_call(
        _combine_body,
        grid=(n // bn,),
        in_specs=in_specs,
        out_specs=pl.BlockSpec((bn, d), lambda i: (i, 0)),
        out_shape=jax.ShapeDtypeStruct((n, d), jnp.float32),
    )(*args)


_SC_MESH = plsc.VectorSubcoreMesh(core_axis_name="c", subcore_axis_name="s")


def _gather(q, kv, ids):
    n, dq = q.shape
    dkv = kv.shape[1]
    nchunks = ids.shape[0]
    e = nchunks * _CHUNK
    tmax = -(-nchunks // _NW)
    ntrips = (tmax + 2 + 3) // 3

    @functools.partial(
        pl.kernel,
        mesh=_SC_MESH,
        out_type=[
            jax.ShapeDtypeStruct((e, dq), q.dtype),
            jax.ShapeDtypeStruct((e, dkv), jnp.int32),
        ],
        scratch_types=[
            pltpu.VMEM((3, 2, _CHUNK), jnp.int32),
            pltpu.VMEM((3, _CHUNK, dq), q.dtype),
            pltpu.VMEM((3, _CHUNK, dkv), jnp.int32),
            pltpu.SemaphoreType.DMA,
            pltpu.SemaphoreType.DMA,
            pltpu.SemaphoreType.DMA,
            pltpu.SemaphoreType.DMA,
            pltpu.SemaphoreType.DMA,
            pltpu.SemaphoreType.DMA,
            pltpu.SemaphoreType.DMA,
            pltpu.SemaphoreType.DMA,
            pltpu.SemaphoreType.DMA,
        ],
    )
    def k(q_hbm, kv_hbm, ids_hbm, qd_out, kvs_out,
          idxb, qbuf, kvbuf, si0, si1, si2, sg0, sg1, sg2, sw0, sw1, sw2):
        w = lax.axis_index("s") * 2 + lax.axis_index("c")
        si = (si0, si1, si2)
        sg = (sg0, sg1, sg2)
        sw = (sw0, sw1, sw2)

        def chunk_of(j):
            return w + _NW * j

        def ok(j):
            return (j >= 0) & (chunk_of(j) < nchunks)

        def issue_idx(j, b):
            @pl.when(ok(j))
            def _():
                pltpu.async_copy(ids_hbm.at[chunk_of(j)], idxb.at[b], si[b])

        issue_idx(0, 0)

        def phase(j, b):
            pb = (b + 2) % 3

            @pl.when(ok(j - 3))
            def _():
                pltpu.make_async_copy(
                    qbuf.at[b], qd_out.at[pl.ds(0, _CHUNK)], sw[b]).wait()
                pltpu.make_async_copy(
                    kvbuf.at[b], kvs_out.at[pl.ds(0, _CHUNK)], sw[b]).wait()

            @pl.when(ok(j))
            def _():
                pltpu.make_async_copy(ids_hbm.at[0], idxb.at[b], si[b]).wait()
                pltpu.async_copy(q_hbm.at[idxb.at[b, 0]], qbuf.at[b], sg[b])
                pltpu.async_copy(kv_hbm.at[idxb.at[b, 1]], kvbuf.at[b], sg[b])

            @pl.when(ok(j - 1))
            def _():
                pltpu.make_async_copy(
                    q_hbm.at[pl.ds(0, _CHUNK)], qbuf.at[pb], sg[pb]).wait()
                pltpu.make_async_copy(
                    kv_hbm.at[pl.ds(0, _CHUNK)], kvbuf.at[pb], sg[pb]).wait()
                c = chunk_of(j - 1)
                pltpu.async_copy(
                    qbuf.at[pb], qd_out.at[pl.ds(c * _CHUNK, _CHUNK)], sw[pb])
                pltpu.async_copy(
                    kvbuf.at[pb], kvs_out.at[pl.ds(c * _CHUNK, _CHUNK)], sw[pb])

            issue_idx(j + 1, (b + 1) % 3)

        def body(jt, carry):
            phase(3 * jt, 0)
            phase(3 * jt + 1, 1)
            phase(3 * jt + 2, 2)
            return carry

        lax.fori_loop(0, ntrips, body, 0)

    return k(q, kv, ids)


def _scatter(message, dst3d, zeros_pd):
    e, d = message.shape
    npad = zeros_pd.shape[0]
    nchunks = dst3d.shape[0]
    tmax = -(-nchunks // _NW)
    rows_per_tile = npad // 16
    ntrips = (tmax + 2 + 3) // 3

    @functools.partial(
        pl.kernel,
        mesh=_SC_MESH,
        out_type=jax.ShapeDtypeStruct((2, npad, d), jnp.float32),
        scratch_types=[
            pltpu.VMEM_SHARED((npad, d), jnp.float32),
            pltpu.VMEM((3, _CHUNK), jnp.int32),
            pltpu.VMEM((3, _CHUNK, d), jnp.float32),
            pltpu.SemaphoreType.DMA,
            pltpu.SemaphoreType.DMA,
            pltpu.SemaphoreType.DMA,
            pltpu.SemaphoreType.DMA,
            pltpu.SemaphoreType.DMA,
            pltpu.SemaphoreType.DMA,
            pltpu.SemaphoreType.DMA,
            pltpu.SemaphoreType.DMA,
            pltpu.SemaphoreType.DMA,
        ],
    )
    def k(msg_hbm, dst_hbm, zero_hbm, part_out, acc_sh, dstb, msgb,
          si0, si1, si2, sl0, sl1, sl2, ss0, ss1, ss2):
        c = lax.axis_index("c")
        s = lax.axis_index("s")
        w = s * 2 + c
        r0 = s * rows_per_tile
        si = (si0, si1, si2)
        sl = (sl0, sl1, sl2)
        ss = (ss0, ss1, ss2)

        pltpu.sync_copy(zero_hbm.at[pl.ds(r0, rows_per_tile)],
                        acc_sh.at[pl.ds(r0, rows_per_tile)])
        plsc.subcore_barrier()

        def chunk_of(j):
            return w + _NW * j

        def ok(j):
            return (j >= 0) & (chunk_of(j) < nchunks)

        def phase(j, b):
            pb = (b + 2) % 3

            @pl.when(ok(j - 3))
            def _():
                pltpu.make_async_copy(
                    msgb.at[b], acc_sh.at[pl.ds(0, _CHUNK)], ss[b]).wait()

            @pl.when(ok(j))
            def _():
                chunk = chunk_of(j)
                pltpu.async_copy(dst_hbm.at[chunk, 0], dstb.at[b], si[b])
                pltpu.async_copy(
                    msg_hbm.at[pl.ds(chunk * _CHUNK, _CHUNK)], msgb.at[b], sl[b])

            @pl.when(ok(j - 1))
            def _():
                pltpu.make_async_copy(
                    dst_hbm.at[0, 0], dstb.at[pb], si[pb]).wait()
                pltpu.make_async_copy(
                    msg_hbm.at[pl.ds(0, _CHUNK)], msgb.at[pb], sl[pb]).wait()
                pltpu.async_copy(
                    msgb.at[pb], acc_sh.at[dstb.at[pb]], ss[pb], add=True)

        def body(jt, carry):
            phase(3 * jt, 0)
            phase(3 * jt + 1, 1)
            phase(3 * jt + 2, 2)
            return carry

        lax.fori_loop(0, ntrips, body, 0)
        plsc.subcore_barrier()
        pltpu.sync_copy(acc_sh.at[pl.ds(r0, rows_per_tile)],
                        part_out.at[c, pl.ds(r0, rows_per_tile)])

    return k(message, dst3d, zeros_pd)


def kernel(atom_feature, edge_feature, edge_index, Wq, bq, Wk, bk, Wv, bv, We, be):
    n, d = atom_feature.shape
    e = edge_feature.shape[0]
    nchunks = e // _CHUNK
    npad = ((n + 127) // 128) * 128
    slice_chunks = (1250, 1250)
    assert sum(slice_chunks) == nchunks

    q, kvp = _node_proj(atom_feature, Wq.T, Wk.T, Wv.T,
                        bq.reshape(1, d), bk.reshape(1, d), bv.reshape(1, d))
    dst3d = edge_index[1].reshape(nchunks, 1, _CHUNK)
    ids = jnp.stack([edge_index[1].reshape(nchunks, _CHUNK),
                     edge_index[0].reshape(nchunks, _CHUNK)], axis=1)
    zeros_pd = jnp.zeros((npad, d), jnp.float32)

    prev_eo = None
    edge_out = None
    atom_out = atom_feature
    c0 = 0
    for cs in slice_chunks:
        ids_s = ids[c0:c0 + cs]
        qd_s, kvs_s = _gather(q, kvp, ids_s)
        edge_out, msg_s = _edge_math(qd_s, kvs_s, edge_feature, We.T,
                                     be.reshape(1, d),
                                     c0 * _CHUNK // 8000, prev_eo)
        prev_eo = edge_out
        partials = _scatter(msg_s, dst3d[c0:c0 + cs], zeros_pd)
        atom_out = _combine(atom_out, [partials])
        c0 += cs
    return (atom_out, edge_out)
